# Initial kernel scaffold; baseline (speedup 1.0000x reference)
import jax
import jax.numpy as jnp
from jax.experimental import pallas as pl


def kernel(x, table, W1, b1, W2, b2, P1, pb1, P2, pb2):
    raise NotImplementedError("write your pallas kernel here")


# speedup vs baseline: 34.6954x; 34.6954x over previous
import functools

import jax
import jax.numpy as jnp
from jax import lax
from jax.experimental import pallas as pl
from jax.experimental.pallas import tpu as pltpu
from jax.experimental.pallas import tpu_sc as plsc

B, S, D = 128, 8192, 64
V = 6
K = int(S * 0.15)
KPAD = 1232
L = 16
NCHUNK = S // L
NC, NS = 2, 16
NW = NC * NS
ROWS_PER_W = B // NW


def _sc_body(x_hbm, tbl_hbm, w1_hbm, w2_hbm, p1_hbm, p2_hbm, bias_hbm,
             idx_hbm, pred_hbm,
             xrow, chunkcnt, outidx, tbl, w1, p1, w2, p2, bias,
             score_s, totals_s, glob_r, nsel_s, pooled_r, pred_r):
    wid = lax.axis_index("s") * NC + lax.axis_index("c")
    iota = lax.iota(jnp.int32, L)

    pltpu.sync_copy(tbl_hbm, tbl)
    pltpu.sync_copy(w1_hbm, w1)
    pltpu.sync_copy(w2_hbm, w2)
    pltpu.sync_copy(p1_hbm, p1)
    pltpu.sync_copy(p2_hbm, p2)
    pltpu.sync_copy(bias_hbm, bias)

    def splat(ref, idx):
        return plsc.load_gather(ref, [jnp.full((L,), idx, jnp.int32)])

    def splat1(ref, idx):
        return plsc.load_gather(ref, [jnp.full((L,), idx + 1, jnp.int32)])

    def store1(ref, vec):
        plsc.store_scatter(ref, [iota + 1], vec)

    sv = jnp.zeros((L,), jnp.float32)
    for v in range(V):
        def mlp_step(d, acc):
            a0, a1 = acc
            t = splat(tbl, v * D + d)
            return (a0 + t * w1[d, pl.ds(0, L)], a1 + t * w1[d, pl.ds(L, L)])
        a0, a1 = lax.fori_loop(
            0, D, mlp_step, (bias[pl.ds(0, L)], bias[pl.ds(L, L)]))
        h0 = jnp.maximum(a0, 0.0)
        h1 = jnp.maximum(a1, 0.0)
        logit = jnp.sum(h0 * w2[pl.ds(0, L)] + h1 * w2[pl.ds(L, L)])
        sv = jnp.where(iota == v, logit, sv)
    store1(score_s, sv)

    predvec = jnp.zeros((L,), jnp.float32)
    for i in range(ROWS_PER_W):
        row = wid * ROWS_PER_W + i
        pltpu.sync_copy(x_hbm.at[row], xrow)

        def zero_step(c, carry):
            chunkcnt[pl.ds(pl.multiple_of(c * L, L), L)] = jnp.zeros(
                (L,), jnp.int32)
            return carry
        lax.fori_loop(0, NCHUNK, zero_step, 0)

        def hist_step(c, carry):
            xv = xrow[pl.ds(pl.multiple_of(c * L, L), L)]
            cntv, lastm = plsc.scan_count(xv)
            plsc.addupdate_scatter(chunkcnt, [c * L + xv], cntv, mask=lastm)
            return carry
        lax.fori_loop(0, NCHUNK, hist_step, 0)

        def prefix_step(c, run):
            off = pl.ds(pl.multiple_of(c * L, L), L)
            v = chunkcnt[off]
            chunkcnt[off] = run
            return run + v
        totals = lax.fori_loop(
            0, NCHUNK, prefix_step, jnp.zeros((L,), jnp.int32))
        store1(totals_s, totals)

        glob = jnp.zeros((L,), jnp.int32)
        for u in range(V):
            su = splat1(score_s, u)
            tu = splat1(totals_s, u)
            before = (su > sv) | ((su == sv) & (u < iota))
            glob = glob + jnp.where(before, tu, 0)
        glob_r[...] = glob
        nsel = jnp.minimum(glob + totals, K) - jnp.minimum(glob, K)
        store1(nsel_s, nsel)

        def scatter_step(c, carry):
            xv = xrow[pl.ds(pl.multiple_of(c * L, L), L)]
            cntv, _lastm = plsc.scan_count(xv)
            cb = plsc.load_gather(chunkcnt, [c * L + xv])
            gb = plsc.load_gather(glob_r, [xv])
            dest = gb + cb + cntv - 1
            pos = iota + c * L
            plsc.store_scatter(outidx, [dest], pos, mask=dest < K)
            return carry
        lax.fori_loop(0, NCHUNK, scatter_step, 0)

        pltpu.sync_copy(outidx, idx_hbm.at[row])

        p = [jnp.zeros((L,), jnp.float32) for _ in range(4)]
        for v in range(V):
            nf = splat1(nsel_s, v).astype(jnp.float32)
            p = [p[j] + nf * tbl[pl.ds(v * D + j * L, L)] for j in range(4)]
        kf = jnp.float32(K)
        for j in range(4):
            pooled_r[pl.ds(j * L, L)] = p[j] / kf

        def head_step(d, acc):
            s = splat(pooled_r, d)
            return tuple(acc[j] + s * p1[d, pl.ds(j * L, L)] for j in range(4))
        acc = lax.fori_loop(
            0, D, head_step,
            tuple(bias[pl.ds(32 + j * L, L)] for j in range(4)))
        ph = [jnp.maximum(a, 0.0) for a in acc]
        t = ph[0] * p2[pl.ds(0, L)]
        for j in range(1, 4):
            t = t + ph[j] * p2[pl.ds(j * L, L)]
        z = jnp.sum(t)
        zv = jnp.full((L,), z, jnp.float32) + splat(bias, 97)
        sig = 1.0 / (1.0 + jnp.exp(-zv))
        predvec = jnp.where(iota == i, sig, predvec)

    pred_r[...] = predvec
    pltpu.sync_copy(pred_r, pred_hbm.at[wid])


@jax.jit
def _run(x, table, W1, W2v, P1, P2v, bias):
    mesh = plsc.VectorSubcoreMesh(core_axis_name="c", subcore_axis_name="s")
    f = pl.kernel(
        _sc_body,
        out_type=(
            jax.ShapeDtypeStruct((B, KPAD), jnp.int32),
            jax.ShapeDtypeStruct((NW, L), jnp.float32),
        ),
        mesh=mesh,
        compiler_params=pltpu.CompilerParams(
            needs_layout_passes=False, use_tc_tiling_on_sc=False),
        scratch_types=[
            pltpu.VMEM((S,), jnp.int32),
            pltpu.VMEM((NCHUNK * L,), jnp.int32),
            pltpu.VMEM((KPAD,), jnp.int32),
            pltpu.VMEM((V * D,), jnp.float32),
            pltpu.VMEM((D, 32), jnp.float32),
            pltpu.VMEM((D, D), jnp.float32),
            pltpu.VMEM((32,), jnp.float32),
            pltpu.VMEM((D,), jnp.float32),
            pltpu.VMEM((128,), jnp.float32),
            pltpu.VMEM((2 * L,), jnp.float32),
            pltpu.VMEM((2 * L,), jnp.int32),
            pltpu.VMEM((L,), jnp.int32),
            pltpu.VMEM((2 * L,), jnp.int32),
            pltpu.VMEM((D,), jnp.float32),
            pltpu.VMEM((L,), jnp.float32),
        ],
    )
    return f(x, table, W1, W2v, P1, P2v, bias)


def kernel(x, table, W1, b1, W2, b2, P1, pb1, P2, pb2):
    bias = jnp.concatenate(
        [b1, pb1, b2, pb2, jnp.zeros((30,), jnp.float32)])
    idx_pad, predbuf = _run(
        x.astype(jnp.int32), table.reshape(V * D), W1, W2.reshape(32), P1,
        P2.reshape(64), bias)
    prediction = predbuf[:, :ROWS_PER_W].reshape(B)
    indices = idx_pad[:, :K]
    return (prediction, indices)


# speedup vs baseline: 77.0091x; 2.2196x over previous
import functools

import jax
import jax.numpy as jnp
from jax import lax
from jax.experimental import pallas as pl
from jax.experimental.pallas import tpu as pltpu
from jax.experimental.pallas import tpu_sc as plsc

B, S, D = 128, 8192, 64
V = 6
K = int(S * 0.15)
KPAD = 1232
L = 16
NCHUNK = S // L
NC, NS = 2, 16
NW = NC * NS
ROWS_PER_W = B // NW


def _sc_body(x_hbm, tbl_hbm, w1_hbm, w2_hbm, p1_hbm, p2_hbm, bias_hbm,
             idx_hbm, pred_hbm,
             xrow, chunkcnt, outidx, tbl, w1, p1, w2, p2, bias,
             score_s, totals_s, glob_r, nsel_s, pooled_r, pred_r):
    wid = lax.axis_index("s") * NC + lax.axis_index("c")
    iota = lax.iota(jnp.int32, L)

    pltpu.sync_copy(tbl_hbm, tbl)
    pltpu.sync_copy(w1_hbm, w1)
    pltpu.sync_copy(w2_hbm, w2)
    pltpu.sync_copy(p1_hbm, p1)
    pltpu.sync_copy(p2_hbm, p2)
    pltpu.sync_copy(bias_hbm, bias)

    def splat(ref, idx):
        return plsc.load_gather(ref, [jnp.full((L,), idx, jnp.int32)])

    def splat1(ref, idx):
        return plsc.load_gather(ref, [jnp.full((L,), idx + 1, jnp.int32)])

    def store1(ref, vec):
        plsc.store_scatter(ref, [iota + 1], vec)

    sv = jnp.zeros((L,), jnp.float32)
    for v in range(V):
        def mlp_step(d, acc):
            a0, a1 = acc
            t = splat(tbl, v * D + d)
            return (a0 + t * w1[d, pl.ds(0, L)], a1 + t * w1[d, pl.ds(L, L)])
        a0, a1 = lax.fori_loop(
            0, D, mlp_step, (bias[pl.ds(0, L)], bias[pl.ds(L, L)]))
        h0 = jnp.maximum(a0, 0.0)
        h1 = jnp.maximum(a1, 0.0)
        logit = jnp.sum(h0 * w2[pl.ds(0, L)] + h1 * w2[pl.ds(L, L)])
        sv = jnp.where(iota == v, logit, sv)
    store1(score_s, sv)

    predvec = jnp.zeros((L,), jnp.float32)
    for i in range(ROWS_PER_W):
        row = wid * ROWS_PER_W + i
        pltpu.sync_copy(x_hbm.at[row], xrow)

        @plsc.parallel_loop(0, NCHUNK, unroll=8)
        def _(c):
            chunkcnt[pl.ds(pl.multiple_of(c * L, L), L)] = jnp.zeros(
                (L,), jnp.int32)

        @plsc.parallel_loop(0, NCHUNK, unroll=4)
        def _(c):
            xv = xrow[pl.ds(pl.multiple_of(c * L, L), L)]
            cntv, lastm = plsc.scan_count(xv)
            plsc.addupdate_scatter(chunkcnt, [c * L + xv], cntv, mask=lastm)

        @plsc.parallel_loop(0, NCHUNK, unroll=2,
                            carry=jnp.zeros((L,), jnp.int32))
        def totals(c, run):
            off = pl.ds(pl.multiple_of(c * L, L), L)
            v = chunkcnt[off]
            chunkcnt[off] = run
            return run + v
        store1(totals_s, totals)

        glob = jnp.zeros((L,), jnp.int32)
        for u in range(V):
            su = splat1(score_s, u)
            tu = splat1(totals_s, u)
            before = (su > sv) | ((su == sv) & (u < iota))
            glob = glob + jnp.where(before, tu, 0)
        glob_r[...] = glob
        nsel = jnp.minimum(glob + totals, K) - jnp.minimum(glob, K)
        store1(nsel_s, nsel)

        @plsc.parallel_loop(0, NCHUNK, unroll=4)
        def _(c):
            xv = xrow[pl.ds(pl.multiple_of(c * L, L), L)]
            cntv, _lastm = plsc.scan_count(xv)
            cb = plsc.load_gather(chunkcnt, [c * L + xv])
            gb = plsc.load_gather(glob_r, [xv])
            dest = gb + cb + cntv - 1
            pos = iota + c * L
            plsc.store_scatter(outidx, [dest], pos, mask=dest < K)

        pltpu.sync_copy(outidx, idx_hbm.at[row])

        p = [jnp.zeros((L,), jnp.float32) for _ in range(4)]
        for v in range(V):
            nf = splat1(nsel_s, v).astype(jnp.float32)
            p = [p[j] + nf * tbl[pl.ds(v * D + j * L, L)] for j in range(4)]
        kf = jnp.float32(K)
        for j in range(4):
            pooled_r[pl.ds(j * L, L)] = p[j] / kf

        def head_step(d, acc):
            s = splat(pooled_r, d)
            return tuple(acc[j] + s * p1[d, pl.ds(j * L, L)] for j in range(4))
        acc = lax.fori_loop(
            0, D, head_step,
            tuple(bias[pl.ds(32 + j * L, L)] for j in range(4)))
        ph = [jnp.maximum(a, 0.0) for a in acc]
        t = ph[0] * p2[pl.ds(0, L)]
        for j in range(1, 4):
            t = t + ph[j] * p2[pl.ds(j * L, L)]
        z = jnp.sum(t)
        zv = jnp.full((L,), z, jnp.float32) + splat(bias, 97)
        sig = 1.0 / (1.0 + jnp.exp(-zv))
        predvec = jnp.where(iota == i, sig, predvec)

    pred_r[...] = predvec
    pltpu.sync_copy(pred_r, pred_hbm.at[wid])


@jax.jit
def _run(x, table, W1, W2v, P1, P2v, bias):
    mesh = plsc.VectorSubcoreMesh(core_axis_name="c", subcore_axis_name="s")
    f = pl.kernel(
        _sc_body,
        out_type=(
            jax.ShapeDtypeStruct((B, KPAD), jnp.int32),
            jax.ShapeDtypeStruct((NW, L), jnp.float32),
        ),
        mesh=mesh,
        compiler_params=pltpu.CompilerParams(
            needs_layout_passes=False, use_tc_tiling_on_sc=False),
        scratch_types=[
            pltpu.VMEM((S,), jnp.int32),
            pltpu.VMEM((NCHUNK * L,), jnp.int32),
            pltpu.VMEM((KPAD,), jnp.int32),
            pltpu.VMEM((V * D,), jnp.float32),
            pltpu.VMEM((D, 32), jnp.float32),
            pltpu.VMEM((D, D), jnp.float32),
            pltpu.VMEM((32,), jnp.float32),
            pltpu.VMEM((D,), jnp.float32),
            pltpu.VMEM((128,), jnp.float32),
            pltpu.VMEM((2 * L,), jnp.float32),
            pltpu.VMEM((2 * L,), jnp.int32),
            pltpu.VMEM((L,), jnp.int32),
            pltpu.VMEM((2 * L,), jnp.int32),
            pltpu.VMEM((D,), jnp.float32),
            pltpu.VMEM((L,), jnp.float32),
        ],
    )
    return f(x, table, W1, W2v, P1, P2v, bias)


def kernel(x, table, W1, b1, W2, b2, P1, pb1, P2, pb2):
    bias = jnp.concatenate(
        [b1, pb1, b2, pb2, jnp.zeros((30,), jnp.float32)])
    idx_pad, predbuf = _run(
        x.astype(jnp.int32), table.reshape(V * D), W1, W2.reshape(32), P1,
        P2.reshape(64), bias)
    prediction = predbuf[:, :ROWS_PER_W].reshape(B)
    indices = idx_pad[:, :K]
    return (prediction, indices)


# speedup vs baseline: 82.5956x; 1.0725x over previous
import functools

import jax
import jax.numpy as jnp
from jax import lax
from jax.experimental import pallas as pl
from jax.experimental.pallas import tpu as pltpu
from jax.experimental.pallas import tpu_sc as plsc

B, S, D = 128, 8192, 64
V = 6
K = int(S * 0.15)
KPAD = 1232
L = 16
NCHUNK = S // L
NC, NS = 2, 16
NW = NC * NS
ROWS_PER_W = B // NW


def _sc_body(x_hbm, tbl_hbm, w1_hbm, w2_hbm, p1_hbm, p2_hbm, bias_hbm,
             idx_hbm, pred_hbm,
             xrow, chunkcnt, outidx, tbl, w1, p1, w2, p2, bias,
             score_s, totals_s, glob_r, nsel_s, pooled_r, pred_r,
             in_sem, out_sem):
    wid = lax.axis_index("s") * NC + lax.axis_index("c")
    iota = lax.iota(jnp.int32, L)

    pltpu.sync_copy(tbl_hbm, tbl)
    pltpu.sync_copy(w1_hbm, w1)
    pltpu.sync_copy(w2_hbm, w2)
    pltpu.sync_copy(p1_hbm, p1)
    pltpu.sync_copy(p2_hbm, p2)
    pltpu.sync_copy(bias_hbm, bias)

    def splat(ref, idx):
        return plsc.load_gather(ref, [jnp.full((L,), idx, jnp.int32)])

    def splat1(ref, idx):
        return plsc.load_gather(ref, [jnp.full((L,), idx + 1, jnp.int32)])

    def store1(ref, vec):
        plsc.store_scatter(ref, [iota + 1], vec)

    sv = jnp.zeros((L,), jnp.float32)
    for v in range(V):
        def mlp_step(d, acc):
            a0, a1 = acc
            t = splat(tbl, v * D + d)
            return (a0 + t * w1[d, pl.ds(0, L)], a1 + t * w1[d, pl.ds(L, L)])
        a0, a1 = lax.fori_loop(
            0, D, mlp_step, (bias[pl.ds(0, L)], bias[pl.ds(L, L)]))
        h0 = jnp.maximum(a0, 0.0)
        h1 = jnp.maximum(a1, 0.0)
        logit = jnp.sum(h0 * w2[pl.ds(0, L)] + h1 * w2[pl.ds(L, L)])
        sv = jnp.where(iota == v, logit, sv)
    store1(score_s, sv)

    @plsc.parallel_loop(0, NCHUNK, unroll=8)
    def _(c):
        chunkcnt[pl.ds(pl.multiple_of(c * L, L), L)] = jnp.zeros(
            (L,), jnp.int32)

    predvec = jnp.zeros((L,), jnp.float32)
    pending_in = pltpu.async_copy(
        x_hbm.at[wid * ROWS_PER_W], xrow.at[0], in_sem)
    pending_out = None
    for i in range(ROWS_PER_W):
        row = wid * ROWS_PER_W + i
        xbuf = xrow.at[i % 2]
        pending_in.wait()
        if i + 1 < ROWS_PER_W:
            pending_in = pltpu.async_copy(
                x_hbm.at[row + 1], xrow.at[(i + 1) % 2], in_sem)

        @plsc.parallel_loop(0, NCHUNK, unroll=8)
        def _(c):
            xv = xbuf[pl.ds(pl.multiple_of(c * L, L), L)]
            cntv, lastm = plsc.scan_count(xv)
            plsc.addupdate_scatter(chunkcnt, [c * L + xv], cntv, mask=lastm)

        @plsc.parallel_loop(0, NCHUNK, unroll=2,
                            carry=jnp.zeros((L,), jnp.int32))
        def totals(c, run):
            off = pl.ds(pl.multiple_of(c * L, L), L)
            v = chunkcnt[off]
            chunkcnt[off] = run
            return run + v
        store1(totals_s, totals)

        glob = jnp.zeros((L,), jnp.int32)
        for u in range(V):
            su = splat1(score_s, u)
            tu = splat1(totals_s, u)
            before = (su > sv) | ((su == sv) & (u < iota))
            glob = glob + jnp.where(before, tu, 0)
        glob_r[...] = glob
        nsel = jnp.minimum(glob + totals, K) - jnp.minimum(glob, K)
        store1(nsel_s, nsel)

        if pending_out is not None:
            pending_out.wait()

        @plsc.parallel_loop(0, NCHUNK, unroll=8)
        def _(c):
            off = pl.ds(pl.multiple_of(c * L, L), L)
            xv = xbuf[off]
            cntv, _lastm = plsc.scan_count(xv)
            cb = plsc.load_gather(chunkcnt, [c * L + xv])
            gb = plsc.load_gather(glob_r, [xv])
            chunkcnt[off] = jnp.zeros((L,), jnp.int32)
            dest = gb + cb + cntv - 1
            pos = iota + c * L
            plsc.store_scatter(outidx, [dest], pos, mask=dest < K)

        pending_out = pltpu.async_copy(outidx, idx_hbm.at[row], out_sem)

        p = [jnp.zeros((L,), jnp.float32) for _ in range(4)]
        for v in range(V):
            nf = splat1(nsel_s, v).astype(jnp.float32)
            p = [p[j] + nf * tbl[pl.ds(v * D + j * L, L)] for j in range(4)]
        kf = jnp.float32(K)
        for j in range(4):
            pooled_r[pl.ds(j * L, L)] = p[j] / kf

        def head_step(d, acc):
            s = splat(pooled_r, d)
            return tuple(acc[j] + s * p1[d, pl.ds(j * L, L)] for j in range(4))
        acc = lax.fori_loop(
            0, D, head_step,
            tuple(bias[pl.ds(32 + j * L, L)] for j in range(4)))
        ph = [jnp.maximum(a, 0.0) for a in acc]
        t = ph[0] * p2[pl.ds(0, L)]
        for j in range(1, 4):
            t = t + ph[j] * p2[pl.ds(j * L, L)]
        z = jnp.sum(t)
        zv = jnp.full((L,), z, jnp.float32) + splat(bias, 97)
        sig = 1.0 / (1.0 + jnp.exp(-zv))
        predvec = jnp.where(iota == i, sig, predvec)

    pending_out.wait()
    pred_r[...] = predvec
    pltpu.sync_copy(pred_r, pred_hbm.at[wid])


@jax.jit
def _run(x, table, W1, W2v, P1, P2v, bias):
    mesh = plsc.VectorSubcoreMesh(core_axis_name="c", subcore_axis_name="s")
    f = pl.kernel(
        _sc_body,
        out_type=(
            jax.ShapeDtypeStruct((B, KPAD), jnp.int32),
            jax.ShapeDtypeStruct((NW, L), jnp.float32),
        ),
        mesh=mesh,
        compiler_params=pltpu.CompilerParams(
            needs_layout_passes=False, use_tc_tiling_on_sc=False),
        scratch_types=[
            pltpu.VMEM((2, S), jnp.int32),
            pltpu.VMEM((NCHUNK * L,), jnp.int32),
            pltpu.VMEM((KPAD,), jnp.int32),
            pltpu.VMEM((V * D,), jnp.float32),
            pltpu.VMEM((D, 32), jnp.float32),
            pltpu.VMEM((D, D), jnp.float32),
            pltpu.VMEM((32,), jnp.float32),
            pltpu.VMEM((D,), jnp.float32),
            pltpu.VMEM((128,), jnp.float32),
            pltpu.VMEM((2 * L,), jnp.float32),
            pltpu.VMEM((2 * L,), jnp.int32),
            pltpu.VMEM((L,), jnp.int32),
            pltpu.VMEM((2 * L,), jnp.int32),
            pltpu.VMEM((D,), jnp.float32),
            pltpu.VMEM((L,), jnp.float32),
            pltpu.SemaphoreType.DMA,
            pltpu.SemaphoreType.DMA,
        ],
    )
    return f(x, table, W1, W2v, P1, P2v, bias)


def kernel(x, table, W1, b1, W2, b2, P1, pb1, P2, pb2):
    bias = jnp.concatenate(
        [b1, pb1, b2, pb2, jnp.zeros((30,), jnp.float32)])
    idx_pad, predbuf = _run(
        x.astype(jnp.int32), table.reshape(V * D), W1, W2.reshape(32), P1,
        P2.reshape(64), bias)
    prediction = predbuf[:, :ROWS_PER_W].reshape(B)
    indices = idx_pad[:, :K]
    return (prediction, indices)


# speedup vs baseline: 89.4275x; 1.0827x over previous
import functools

import jax
import jax.numpy as jnp
from jax import lax
from jax.experimental import pallas as pl
from jax.experimental.pallas import tpu as pltpu
from jax.experimental.pallas import tpu_sc as plsc

B, S, D = 128, 8192, 64
V = 6
K = int(S * 0.15)
KPAD = 1232
L = 16
NCHUNK = S // L
NC, NS = 2, 16
NW = NC * NS
ROWS_PER_W = B // NW


def _sc_body(x_hbm, tbl_hbm, w1_hbm, w2_hbm, p1_hbm, p2_hbm, bias_hbm,
             idx_hbm, pred_hbm,
             xrow, chunkcnt, outidx, tbl, w1, p1, w2, p2, bias,
             score_s, totals_s, glob_r, nsel_s, pooled_r, pred_r,
             in_sem, out_sem):
    wid = lax.axis_index("s") * NC + lax.axis_index("c")
    iota = lax.iota(jnp.int32, L)

    pltpu.sync_copy(tbl_hbm, tbl)
    pltpu.sync_copy(w1_hbm, w1)
    pltpu.sync_copy(w2_hbm, w2)
    pltpu.sync_copy(p1_hbm, p1)
    pltpu.sync_copy(p2_hbm, p2)
    pltpu.sync_copy(bias_hbm, bias)

    def splat(ref, idx):
        return plsc.load_gather(ref, [jnp.full((L,), idx, jnp.int32)])

    def splat1(ref, idx):
        return plsc.load_gather(ref, [jnp.full((L,), idx + 1, jnp.int32)])

    def store1(ref, vec):
        plsc.store_scatter(ref, [iota + 1], vec)

    sv = jnp.zeros((L,), jnp.float32)
    for v in range(V):
        def mlp_step(d, acc):
            a0, a1 = acc
            t = splat(tbl, v * D + d)
            return (a0 + t * w1[d, pl.ds(0, L)], a1 + t * w1[d, pl.ds(L, L)])
        a0, a1 = lax.fori_loop(
            0, D, mlp_step, (bias[pl.ds(0, L)], bias[pl.ds(L, L)]))
        h0 = jnp.maximum(a0, 0.0)
        h1 = jnp.maximum(a1, 0.0)
        logit = jnp.sum(h0 * w2[pl.ds(0, L)] + h1 * w2[pl.ds(L, L)])
        sv = jnp.where(iota == v, logit, sv)
    store1(score_s, sv)

    @plsc.parallel_loop(0, NCHUNK, unroll=8)
    def _(c):
        chunkcnt[pl.ds(pl.multiple_of(c * L, L), L)] = jnp.zeros(
            (L,), jnp.int32)

    predvec = jnp.zeros((L,), jnp.float32)
    pending_in = pltpu.async_copy(
        x_hbm.at[wid * ROWS_PER_W], xrow.at[0], in_sem)
    pending_out = None
    for i in range(ROWS_PER_W):
        row = wid * ROWS_PER_W + i
        xbuf = xrow.at[i % 2]
        pending_in.wait()
        if i + 1 < ROWS_PER_W:
            pending_in = pltpu.async_copy(
                x_hbm.at[row + 1], xrow.at[(i + 1) % 2], in_sem)

        @plsc.parallel_loop(0, NCHUNK, unroll=8)
        def _(c):
            xv = xbuf[pl.ds(pl.multiple_of(c * L, L), L)]
            cntv, lastm = plsc.scan_count(xv)
            plsc.addupdate_scatter(chunkcnt, [c * L + xv], cntv, mask=lastm)

        @plsc.parallel_loop(0, NCHUNK, unroll=8,
                            carry=jnp.zeros((L,), jnp.int32))
        def totals(c, run):
            off = pl.ds(pl.multiple_of(c * L, L), L)
            v = chunkcnt[off]
            chunkcnt[off] = run
            return run + v
        store1(totals_s, totals)

        glob = jnp.zeros((L,), jnp.int32)
        for u in range(V):
            su = splat1(score_s, u)
            tu = splat1(totals_s, u)
            before = (su > sv) | ((su == sv) & (u < iota))
            glob = glob + jnp.where(before, tu, 0)
        glob_r[...] = glob
        nsel = jnp.minimum(glob + totals, K) - jnp.minimum(glob, K)
        store1(nsel_s, nsel)

        if pending_out is not None:
            pending_out.wait()

        @plsc.parallel_loop(0, NCHUNK, unroll=8)
        def _(c):
            off = pl.ds(pl.multiple_of(c * L, L), L)
            xv = xbuf[off]
            cntv, _lastm = plsc.scan_count(xv)
            cb = plsc.load_gather(chunkcnt, [c * L + xv])
            gb = plsc.load_gather(glob_r, [xv])
            chunkcnt[off] = jnp.zeros((L,), jnp.int32)
            dest = gb + cb + cntv - 1
            pos = iota + c * L
            plsc.store_scatter(outidx, [dest], pos, mask=dest < K)

        pending_out = pltpu.async_copy(outidx, idx_hbm.at[row], out_sem)

        p = [jnp.zeros((L,), jnp.float32) for _ in range(4)]
        for v in range(V):
            nf = splat1(nsel_s, v).astype(jnp.float32)
            p = [p[j] + nf * tbl[pl.ds(v * D + j * L, L)] for j in range(4)]
        kf = jnp.float32(K)
        for j in range(4):
            pooled_r[pl.ds(j * L, L)] = p[j] / kf

        def head_step(d, acc):
            s = splat(pooled_r, d)
            return tuple(acc[j] + s * p1[d, pl.ds(j * L, L)] for j in range(4))
        acc = lax.fori_loop(
            0, D, head_step,
            tuple(bias[pl.ds(32 + j * L, L)] for j in range(4)))
        ph = [jnp.maximum(a, 0.0) for a in acc]
        t = ph[0] * p2[pl.ds(0, L)]
        for j in range(1, 4):
            t = t + ph[j] * p2[pl.ds(j * L, L)]
        z = jnp.sum(t)
        zv = jnp.full((L,), z, jnp.float32) + splat(bias, 97)
        sig = 1.0 / (1.0 + jnp.exp(-zv))
        predvec = jnp.where(iota == i, sig, predvec)

    pending_out.wait()
    pred_r[...] = predvec
    pltpu.sync_copy(pred_r, pred_hbm.at[wid])


@jax.jit
def _run(x, table, W1, W2v, P1, P2v, bias):
    mesh = plsc.VectorSubcoreMesh(core_axis_name="c", subcore_axis_name="s")
    f = pl.kernel(
        _sc_body,
        out_type=(
            jax.ShapeDtypeStruct((B, KPAD), jnp.int32),
            jax.ShapeDtypeStruct((NW, L), jnp.float32),
        ),
        mesh=mesh,
        compiler_params=pltpu.CompilerParams(
            needs_layout_passes=False, use_tc_tiling_on_sc=False),
        scratch_types=[
            pltpu.VMEM((2, S), jnp.int32),
            pltpu.VMEM((NCHUNK * L,), jnp.int32),
            pltpu.VMEM((KPAD,), jnp.int32),
            pltpu.VMEM((V * D,), jnp.float32),
            pltpu.VMEM((D, 32), jnp.float32),
            pltpu.VMEM((D, D), jnp.float32),
            pltpu.VMEM((32,), jnp.float32),
            pltpu.VMEM((D,), jnp.float32),
            pltpu.VMEM((128,), jnp.float32),
            pltpu.VMEM((2 * L,), jnp.float32),
            pltpu.VMEM((2 * L,), jnp.int32),
            pltpu.VMEM((L,), jnp.int32),
            pltpu.VMEM((2 * L,), jnp.int32),
            pltpu.VMEM((D,), jnp.float32),
            pltpu.VMEM((L,), jnp.float32),
            pltpu.SemaphoreType.DMA,
            pltpu.SemaphoreType.DMA,
        ],
    )
    return f(x, table, W1, W2v, P1, P2v, bias)


def kernel(x, table, W1, b1, W2, b2, P1, pb1, P2, pb2):
    bias = jnp.concatenate(
        [b1, pb1, b2, pb2, jnp.zeros((30,), jnp.float32)])
    idx_pad, predbuf = _run(
        x.astype(jnp.int32), table.reshape(V * D), W1, W2.reshape(32), P1,
        P2.reshape(64), bias)
    prediction = predbuf[:, :ROWS_PER_W].reshape(B)
    indices = idx_pad[:, :K]
    return (prediction, indices)
